# Initial kernel scaffold; baseline (speedup 1.0000x reference)
import jax
import jax.numpy as jnp
from jax.experimental import pallas as pl


def kernel(feat_table, W, nodes, neigh_idx):
    raise NotImplementedError("write your pallas kernel here")


# speedup vs baseline: 1.1839x; 1.1839x over previous
import functools

import jax
import jax.numpy as jnp
from jax import lax
from jax.experimental import pallas as pl
from jax.experimental.pallas import tpu as pltpu
from jax.experimental.pallas import tpu_sc as plsc

_NC = 2
_NS = 16
_NW = _NC * _NS

_CHUNK = 128


def _sc_gather(n_per_worker, n_chunks, num_sample, feat_dim, b_pad, table_rows):
    mesh = plsc.VectorSubcoreMesh(core_axis_name="c", subcore_axis_name="s")

    @functools.partial(
        pl.kernel,
        out_type=(
            jax.ShapeDtypeStruct((b_pad, feat_dim), jnp.float32),
            jax.ShapeDtypeStruct((b_pad, feat_dim), jnp.float32),
        ),
        mesh=mesh,
        scratch_types=[
            pltpu.VMEM((_CHUNK,), jnp.int32),
            pltpu.VMEM((num_sample, _CHUNK), jnp.int32),
            pltpu.VMEM((_CHUNK, feat_dim), jnp.float32),
            pltpu.VMEM((_CHUNK, feat_dim), jnp.float32),
            pltpu.SemaphoreType.DMA,
            pltpu.SemaphoreType.DMA,
        ],
    )
    def sc_kernel(nodes_hbm, neigh_t_hbm, table_hbm, self_hbm, sum_hbm,
                  idx_v, nidx_v, self_v, acc_v, sem_self, sem_n):
        wid = lax.axis_index("s") * _NC + lax.axis_index("c")
        base = wid * n_per_worker

        def chunk_body(k, carry):
            start = base + k * _CHUNK
            pltpu.sync_copy(nodes_hbm.at[pl.ds(start, _CHUNK)], idx_v)
            pltpu.sync_copy(neigh_t_hbm.at[:, pl.ds(start, _CHUNK)], nidx_v)
            cp_self = pltpu.async_copy(table_hbm.at[idx_v], self_v, sem_self)
            pltpu.async_copy(table_hbm.at[nidx_v.at[0]], acc_v, sem_n).wait()
            for j in range(1, num_sample):
                pltpu.async_copy(
                    table_hbm.at[nidx_v.at[j]], acc_v, sem_n, add=True
                ).wait()
            cp_self.wait()
            pltpu.sync_copy(self_v, self_hbm.at[pl.ds(start, _CHUNK)])
            pltpu.sync_copy(acc_v, sum_hbm.at[pl.ds(start, _CHUNK)])
            return carry

        lax.fori_loop(0, n_chunks, chunk_body, 0)

    return sc_kernel


def _tc_matmul_kernel(w1_ref, w2_ref, x1_ref, x2_ref, o_ref):
    a = lax.dot_general(
        w1_ref[...], x1_ref[...], (((1,), (1,)), ((), ())),
        preferred_element_type=jnp.float32,
    )
    b = lax.dot_general(
        w2_ref[...], x2_ref[...], (((1,), (1,)), ((), ())),
        preferred_element_type=jnp.float32,
    )
    o_ref[...] = jnp.maximum(a + b, 0.0)


def kernel(feat_table, W, nodes, neigh_idx):
    n_nodes, feat_dim = feat_table.shape
    batch = nodes.shape[0]
    num_sample = neigh_idx.shape[1]
    embed_dim = W.shape[0]

    per_worker_quantum = _NW * _CHUNK
    b_pad = ((batch + per_worker_quantum - 1) // per_worker_quantum) * per_worker_quantum
    n_per_worker = b_pad // _NW
    n_chunks = n_per_worker // _CHUNK

    nodes_p = jnp.pad(nodes.astype(jnp.int32), (0, b_pad - batch))
    neigh_t = jnp.pad(
        neigh_idx.astype(jnp.int32), ((0, b_pad - batch), (0, 0))
    ).T

    sc = _sc_gather(n_per_worker, n_chunks, num_sample, feat_dim, b_pad, n_nodes)
    self_feats, neigh_sum = sc(nodes_p, neigh_t, feat_table)

    w1 = W[:, :feat_dim]
    w2 = W[:, feat_dim:] * (1.0 / num_sample)

    bn = 512
    grid = (b_pad // bn,)
    out = pl.pallas_call(
        _tc_matmul_kernel,
        grid=grid,
        in_specs=[
            pl.BlockSpec((embed_dim, feat_dim), lambda i: (0, 0)),
            pl.BlockSpec((embed_dim, feat_dim), lambda i: (0, 0)),
            pl.BlockSpec((bn, feat_dim), lambda i: (i, 0)),
            pl.BlockSpec((bn, feat_dim), lambda i: (i, 0)),
        ],
        out_specs=pl.BlockSpec((embed_dim, bn), lambda i: (0, i)),
        out_shape=jax.ShapeDtypeStruct((embed_dim, b_pad), jnp.float32),
    )(w1, w2, self_feats, neigh_sum)

    return out[:, :batch]


# speedup vs baseline: 1.2058x; 1.0184x over previous
import functools

import jax
import jax.numpy as jnp
from jax import lax
from jax.experimental import pallas as pl
from jax.experimental.pallas import tpu as pltpu
from jax.experimental.pallas import tpu_sc as plsc

_NC = 2
_NS = 16
_NW = _NC * _NS

_CHUNK = 128


def _sc_gather(n_per_worker, n_chunks, num_sample, feat_dim, b_pad, table_rows):
    mesh = plsc.VectorSubcoreMesh(core_axis_name="c", subcore_axis_name="s")

    @functools.partial(
        pl.kernel,
        out_type=(
            jax.ShapeDtypeStruct((b_pad, feat_dim), jnp.float32),
            jax.ShapeDtypeStruct((b_pad, feat_dim), jnp.float32),
        ),
        mesh=mesh,
        scratch_types=[
            pltpu.VMEM((_CHUNK,), jnp.int32),
            pltpu.VMEM((num_sample, _CHUNK), jnp.int32),
            pltpu.VMEM((_CHUNK, feat_dim), jnp.float32),
            pltpu.VMEM((_CHUNK, feat_dim), jnp.float32),
            pltpu.SemaphoreType.DMA,
            pltpu.SemaphoreType.DMA,
        ],
    )
    def sc_kernel(nodes_hbm, neigh_t_hbm, table_hbm, self_hbm, sum_hbm,
                  idx_v, nidx_v, self_v, acc_v, sem_self, sem_n):
        wid = lax.axis_index("s") * _NC + lax.axis_index("c")
        base = wid * n_per_worker

        def chunk_body(k, carry):
            start = base + k * _CHUNK
            pltpu.sync_copy(nodes_hbm.at[pl.ds(start, _CHUNK)], idx_v)
            pltpu.sync_copy(neigh_t_hbm.at[:, pl.ds(start, _CHUNK)], nidx_v)
            cp_self = pltpu.async_copy(table_hbm.at[idx_v], self_v, sem_self)
            pltpu.async_copy(table_hbm.at[nidx_v.at[0]], acc_v, sem_n).wait()
            adds = [
                pltpu.async_copy(table_hbm.at[nidx_v.at[j]], acc_v, sem_n, add=True)
                for j in range(1, num_sample)
            ]
            for cp in adds:
                cp.wait()
            cp_self.wait()
            pltpu.sync_copy(self_v, self_hbm.at[pl.ds(start, _CHUNK)])
            pltpu.sync_copy(acc_v, sum_hbm.at[pl.ds(start, _CHUNK)])
            return carry

        lax.fori_loop(0, n_chunks, chunk_body, 0)

    return sc_kernel


def _tc_matmul_kernel(w1_ref, w2_ref, x1_ref, x2_ref, o_ref):
    a = lax.dot_general(
        w1_ref[...], x1_ref[...], (((1,), (1,)), ((), ())),
        preferred_element_type=jnp.float32,
    )
    b = lax.dot_general(
        w2_ref[...], x2_ref[...], (((1,), (1,)), ((), ())),
        preferred_element_type=jnp.float32,
    )
    o_ref[...] = jnp.maximum(a + b, 0.0)


def kernel(feat_table, W, nodes, neigh_idx):
    n_nodes, feat_dim = feat_table.shape
    batch = nodes.shape[0]
    num_sample = neigh_idx.shape[1]
    embed_dim = W.shape[0]

    per_worker_quantum = _NW * _CHUNK
    b_pad = ((batch + per_worker_quantum - 1) // per_worker_quantum) * per_worker_quantum
    n_per_worker = b_pad // _NW
    n_chunks = n_per_worker // _CHUNK

    nodes_p = jnp.pad(nodes.astype(jnp.int32), (0, b_pad - batch))
    neigh_t = jnp.pad(
        neigh_idx.astype(jnp.int32), ((0, b_pad - batch), (0, 0))
    ).T

    sc = _sc_gather(n_per_worker, n_chunks, num_sample, feat_dim, b_pad, n_nodes)
    self_feats, neigh_sum = sc(nodes_p, neigh_t, feat_table)

    w1 = W[:, :feat_dim]
    w2 = W[:, feat_dim:] * (1.0 / num_sample)

    bn = 512
    grid = (b_pad // bn,)
    out = pl.pallas_call(
        _tc_matmul_kernel,
        grid=grid,
        in_specs=[
            pl.BlockSpec((embed_dim, feat_dim), lambda i: (0, 0)),
            pl.BlockSpec((embed_dim, feat_dim), lambda i: (0, 0)),
            pl.BlockSpec((bn, feat_dim), lambda i: (i, 0)),
            pl.BlockSpec((bn, feat_dim), lambda i: (i, 0)),
        ],
        out_specs=pl.BlockSpec((embed_dim, bn), lambda i: (0, i)),
        out_shape=jax.ShapeDtypeStruct((embed_dim, b_pad), jnp.float32),
    )(w1, w2, self_feats, neigh_sum)

    return out[:, :batch]


# speedup vs baseline: 3.2205x; 2.6709x over previous
import functools

import jax
import jax.numpy as jnp
from jax import lax
from jax.experimental import pallas as pl
from jax.experimental.pallas import tpu as pltpu
from jax.experimental.pallas import tpu_sc as plsc

_NC = 2
_NS = 16
_NW = _NC * _NS

_WIN = 32
_L = 16


def _sc_gather(n_per_worker, n_windows, num_sample, feat_dim, b_pad):
    mesh = plsc.VectorSubcoreMesh(core_axis_name="c", subcore_axis_name="s")
    rowsz = feat_dim
    nbuf_sz = _WIN * num_sample * rowsz
    obuf_sz = _WIN * rowsz

    @functools.partial(
        pl.kernel,
        out_type=(
            jax.ShapeDtypeStruct((b_pad * feat_dim,), jnp.float32),
            jax.ShapeDtypeStruct((b_pad * feat_dim,), jnp.float32),
        ),
        mesh=mesh,
        scratch_types=[
            pltpu.VMEM((_WIN,), jnp.int32),
            pltpu.VMEM((num_sample * _WIN,), jnp.int32),
            pltpu.VMEM((nbuf_sz,), jnp.float32),
            pltpu.VMEM((obuf_sz,), jnp.float32),
            pltpu.VMEM((obuf_sz,), jnp.float32),
            pltpu.SemaphoreType.DMA,
        ],
    )
    def sc_kernel(nodes_hbm, narr_hbm, tab_hbm, self_hbm, sum_hbm,
                  nodes_iv, nidx_iv, nrows_v, self_v, acc_v,
                  sem_g):
        wid = lax.axis_index("s") * _NC + lax.axis_index("c")
        base = wid * n_per_worker

        def fire(w):
            row0 = base + w * _WIN
            pltpu.sync_copy(nodes_hbm.at[pl.ds(row0, _WIN)], nodes_iv)
            pltpu.sync_copy(
                narr_hbm.at[pl.ds((wid * n_windows + w) * num_sample * _WIN,
                                  num_sample * _WIN)],
                nidx_iv)

            def group_fire(g, carry):
                nv = nodes_iv[pl.ds(g * _L, _L)]
                for i in range(_L):
                    s = nv[i] * rowsz
                    pltpu.async_copy(
                        tab_hbm.at[pl.ds(s, rowsz)],
                        self_v.at[pl.ds((g * _L + i) * rowsz, rowsz)], sem_g)
                for j in range(num_sample):
                    tv = nidx_iv[pl.ds(j * _WIN + g * _L, _L)]
                    for i in range(_L):
                        t = tv[i] * rowsz
                        pltpu.async_copy(
                            tab_hbm.at[pl.ds(t, rowsz)],
                            nrows_v.at[pl.ds(
                                ((g * _L + i) * num_sample + j) * rowsz,
                                rowsz)],
                            sem_g)
                return carry

            lax.fori_loop(0, _WIN // _L, group_fire, 0)

        def drain():
            pltpu.make_async_copy(
                tab_hbm.at[pl.ds(0, nbuf_sz)], nrows_v, sem_g).wait()
            pltpu.make_async_copy(
                tab_hbm.at[pl.ds(0, obuf_sz)], self_v, sem_g).wait()

        def reduce_write(w):
            def node_red(i, carry):
                noff = i * num_sample * rowsz
                for c in range(rowsz // _L):
                    off = noff + c * _L
                    v = nrows_v[pl.ds(off, _L)]
                    for j in range(1, num_sample):
                        v = v + nrows_v[pl.ds(off + j * rowsz, _L)]
                    acc_v[pl.ds(i * rowsz + c * _L, _L)] = v
                return carry

            lax.fori_loop(0, _WIN, node_red, 0)
            row0 = base + w * _WIN
            pltpu.sync_copy(self_v, self_hbm.at[pl.ds(row0 * rowsz, obuf_sz)])
            pltpu.sync_copy(acc_v, sum_hbm.at[pl.ds(row0 * rowsz, obuf_sz)])

        def window_body(w, carry):
            fire(w)
            drain()
            reduce_write(w)
            return carry

        lax.fori_loop(0, n_windows, window_body, 0)

    return sc_kernel


def _tc_matmul_kernel(w1_ref, w2_ref, x1_ref, x2_ref, o_ref):
    a = lax.dot_general(
        w1_ref[...], x1_ref[...], (((1,), (1,)), ((), ())),
        preferred_element_type=jnp.float32,
    )
    b = lax.dot_general(
        w2_ref[...], x2_ref[...], (((1,), (1,)), ((), ())),
        preferred_element_type=jnp.float32,
    )
    o_ref[...] = jnp.maximum(a + b, 0.0)


def kernel(feat_table, W, nodes, neigh_idx):
    n_nodes, feat_dim = feat_table.shape
    batch = nodes.shape[0]
    num_sample = neigh_idx.shape[1]
    embed_dim = W.shape[0]

    quantum = _NW * _WIN
    b_pad = ((batch + quantum - 1) // quantum) * quantum
    n_per_worker = b_pad // _NW
    n_windows = n_per_worker // _WIN

    nodes_p = jnp.pad(nodes.astype(jnp.int32), (0, b_pad - batch))
    narr = (
        jnp.pad(neigh_idx.astype(jnp.int32), ((0, b_pad - batch), (0, 0)))
        .reshape(_NW * n_windows, _WIN, num_sample)
        .transpose(0, 2, 1)
        .reshape(-1)
    )

    sc = _sc_gather(n_per_worker, n_windows, num_sample, feat_dim, b_pad)
    self_flat, sum_flat = sc(nodes_p, narr, feat_table.reshape(-1))
    self_feats = self_flat.reshape(b_pad, feat_dim)
    neigh_sum = sum_flat.reshape(b_pad, feat_dim)

    w1 = W[:, :feat_dim]
    w2 = W[:, feat_dim:] * (1.0 / num_sample)

    bn = 512
    grid = (b_pad // bn,)
    out = pl.pallas_call(
        _tc_matmul_kernel,
        grid=grid,
        in_specs=[
            pl.BlockSpec((embed_dim, feat_dim), lambda i: (0, 0)),
            pl.BlockSpec((embed_dim, feat_dim), lambda i: (0, 0)),
            pl.BlockSpec((bn, feat_dim), lambda i: (i, 0)),
            pl.BlockSpec((bn, feat_dim), lambda i: (i, 0)),
        ],
        out_specs=pl.BlockSpec((embed_dim, bn), lambda i: (0, i)),
        out_shape=jax.ShapeDtypeStruct((embed_dim, b_pad), jnp.float32),
    )(w1, w2, self_feats, neigh_sum)

    return out[:, :batch]


# speedup vs baseline: 3.8279x; 1.1886x over previous
import functools

import jax
import jax.numpy as jnp
from jax import lax
from jax.experimental import pallas as pl
from jax.experimental.pallas import tpu as pltpu
from jax.experimental.pallas import tpu_sc as plsc

_NC = 2
_NS = 16
_NW = _NC * _NS

_WIN = 32
_L = 16


def _sc_gather(n_per_worker, n_windows, num_sample, feat_dim, b_pad):
    mesh = plsc.VectorSubcoreMesh(core_axis_name="c", subcore_axis_name="s")
    rowsz = feat_dim
    nbuf_sz = _WIN * num_sample * rowsz
    obuf_sz = _WIN * rowsz

    assert n_windows % 2 == 1 and n_windows >= 3

    @functools.partial(
        pl.kernel,
        out_type=(
            jax.ShapeDtypeStruct((b_pad * feat_dim,), jnp.float32),
            jax.ShapeDtypeStruct((b_pad * feat_dim,), jnp.float32),
        ),
        mesh=mesh,
        scratch_types=[
            pltpu.VMEM((_WIN,), jnp.int32),
            pltpu.VMEM((_WIN,), jnp.int32),
            pltpu.VMEM((num_sample * _WIN,), jnp.int32),
            pltpu.VMEM((num_sample * _WIN,), jnp.int32),
            pltpu.VMEM((nbuf_sz,), jnp.float32),
            pltpu.VMEM((nbuf_sz,), jnp.float32),
            pltpu.VMEM((obuf_sz,), jnp.float32),
            pltpu.VMEM((obuf_sz,), jnp.float32),
            pltpu.VMEM((obuf_sz,), jnp.float32),
            pltpu.SemaphoreType.DMA,
            pltpu.SemaphoreType.DMA,
            pltpu.SemaphoreType.DMA,
        ],
    )
    def sc_kernel(nodes_hbm, narr_hbm, tab_hbm, self_hbm, sum_hbm,
                  nod0, nod1, nid0, nid1, rows0, rows1, self0, self1, acc_v,
                  sem_a, sem_b, sem_i):
        wid = lax.axis_index("s") * _NC + lax.axis_index("c")
        base = wid * n_per_worker

        def fire_idx(w, nod_iv, nid_iv):
            pltpu.async_copy(
                nodes_hbm.at[pl.ds(base + w * _WIN, _WIN)], nod_iv, sem_i)
            pltpu.async_copy(
                narr_hbm.at[pl.ds((wid * n_windows + w) * num_sample * _WIN,
                                  num_sample * _WIN)],
                nid_iv, sem_i)

        def wait_idx(nod_iv, nid_iv):
            pltpu.make_async_copy(
                nodes_hbm.at[pl.ds(0, _WIN)], nod_iv, sem_i).wait()
            pltpu.make_async_copy(
                narr_hbm.at[pl.ds(0, num_sample * _WIN)], nid_iv, sem_i).wait()

        def fire_gather(nod_iv, nid_iv, nrows_v, self_v, sem):
            def group_fire(g, carry):
                nv = nod_iv[pl.ds(g * _L, _L)]
                for i in range(_L):
                    s = nv[i] * rowsz
                    pltpu.async_copy(
                        tab_hbm.at[pl.ds(s, rowsz)],
                        self_v.at[pl.ds((g * _L + i) * rowsz, rowsz)], sem)
                for j in range(num_sample):
                    tv = nid_iv[pl.ds(j * _WIN + g * _L, _L)]
                    for i in range(_L):
                        t = tv[i] * rowsz
                        pltpu.async_copy(
                            tab_hbm.at[pl.ds(t, rowsz)],
                            nrows_v.at[pl.ds(
                                ((g * _L + i) * num_sample + j) * rowsz,
                                rowsz)],
                            sem)
                return carry

            lax.fori_loop(0, _WIN // _L, group_fire, 0)

        def drain_gather(nrows_v, self_v, sem):
            pltpu.make_async_copy(
                tab_hbm.at[pl.ds(0, nbuf_sz)], nrows_v, sem).wait()
            pltpu.make_async_copy(
                tab_hbm.at[pl.ds(0, obuf_sz)], self_v, sem).wait()

        def reduce_write(w, nrows_v, self_v):
            def node_red(i, carry):
                noff = i * num_sample * rowsz
                for c in range(rowsz // _L):
                    off = noff + c * _L
                    v = nrows_v[pl.ds(off, _L)]
                    for j in range(1, num_sample):
                        v = v + nrows_v[pl.ds(off + j * rowsz, _L)]
                    acc_v[pl.ds(i * rowsz + c * _L, _L)] = v
                return carry

            lax.fori_loop(0, _WIN, node_red, 0)
            row0 = base + w * _WIN
            pltpu.sync_copy(self_v, self_hbm.at[pl.ds(row0 * rowsz, obuf_sz)])
            pltpu.sync_copy(acc_v, sum_hbm.at[pl.ds(row0 * rowsz, obuf_sz)])

        fire_idx(0, nod0, nid0)
        wait_idx(nod0, nid0)
        fire_gather(nod0, nid0, rows0, self0, sem_a)
        fire_idx(1, nod1, nid1)

        def pair_body(p, carry):
            w0 = 2 * p
            w3 = w0 + 3
            wait_idx(nod1, nid1)
            fire_gather(nod1, nid1, rows1, self1, sem_b)
            fire_idx(w0 + 2, nod0, nid0)
            drain_gather(rows0, self0, sem_a)
            reduce_write(w0, rows0, self0)
            wait_idx(nod0, nid0)
            fire_gather(nod0, nid0, rows0, self0, sem_a)

            @pl.when(w3 < n_windows)
            def _():
                fire_idx(w3, nod1, nid1)

            drain_gather(rows1, self1, sem_b)
            reduce_write(w0 + 1, rows1, self1)
            return carry

        lax.fori_loop(0, (n_windows - 1) // 2, pair_body, 0)
        drain_gather(rows0, self0, sem_a)
        reduce_write(n_windows - 1, rows0, self0)

    return sc_kernel


def _tc_matmul_kernel(w1_ref, w2_ref, x1_ref, x2_ref, o_ref):
    a = lax.dot_general(
        w1_ref[...], x1_ref[...], (((1,), (1,)), ((), ())),
        preferred_element_type=jnp.float32,
    )
    b = lax.dot_general(
        w2_ref[...], x2_ref[...], (((1,), (1,)), ((), ())),
        preferred_element_type=jnp.float32,
    )
    o_ref[...] = jnp.maximum(a + b, 0.0)


def kernel(feat_table, W, nodes, neigh_idx):
    n_nodes, feat_dim = feat_table.shape
    batch = nodes.shape[0]
    num_sample = neigh_idx.shape[1]
    embed_dim = W.shape[0]

    quantum = _NW * _WIN
    b_pad = ((batch + quantum - 1) // quantum) * quantum
    if (b_pad // quantum) % 2 == 0:
        b_pad += quantum
    n_per_worker = b_pad // _NW
    n_windows = n_per_worker // _WIN

    nodes_p = jnp.pad(nodes.astype(jnp.int32), (0, b_pad - batch))
    narr = (
        jnp.pad(neigh_idx.astype(jnp.int32), ((0, b_pad - batch), (0, 0)))
        .reshape(_NW * n_windows, _WIN, num_sample)
        .transpose(0, 2, 1)
        .reshape(-1)
    )

    sc = _sc_gather(n_per_worker, n_windows, num_sample, feat_dim, b_pad)
    self_flat, sum_flat = sc(nodes_p, narr, feat_table.reshape(-1))
    self_feats = self_flat.reshape(b_pad, feat_dim)
    neigh_sum = sum_flat.reshape(b_pad, feat_dim)

    w1 = W[:, :feat_dim]
    w2 = W[:, feat_dim:] * (1.0 / num_sample)

    bn = 512
    grid = (b_pad // bn,)
    out = pl.pallas_call(
        _tc_matmul_kernel,
        grid=grid,
        in_specs=[
            pl.BlockSpec((embed_dim, feat_dim), lambda i: (0, 0)),
            pl.BlockSpec((embed_dim, feat_dim), lambda i: (0, 0)),
            pl.BlockSpec((bn, feat_dim), lambda i: (i, 0)),
            pl.BlockSpec((bn, feat_dim), lambda i: (i, 0)),
        ],
        out_specs=pl.BlockSpec((embed_dim, bn), lambda i: (0, i)),
        out_shape=jax.ShapeDtypeStruct((embed_dim, b_pad), jnp.float32),
    )(w1, w2, self_feats, neigh_sum)

    return out[:, :batch]
